# baseline (speedup 1.0000x reference)
import functools

import jax
import jax.numpy as jnp
from jax.experimental import pallas as pl
from jax.experimental.pallas import tpu as pltpu

_NUM_SEG = 256
_N = 100000
_D = 128
_BLK = 10000
_NBLK = _N // _BLK


def _body(x_ref, bt_ref, w1_ref, b1_ref, w2_ref, b2_ref, out_ref,
          s_acc, d_acc):
    i = pl.program_id(0)

    @pl.when(i == 0)
    def _init():
        s_acc[...] = jnp.zeros_like(s_acc)
        d_acc[...] = jnp.zeros_like(d_acc)

    x = x_ref[...]
    xb = x.astype(jnp.bfloat16)
    h = jnp.tanh(
        jnp.dot(xb, w1_ref[...].astype(jnp.bfloat16),
                preferred_element_type=jnp.float32)
        + b1_ref[...])
    s = (jnp.sum(h * w2_ref[...], axis=1, keepdims=True)
         + b2_ref[...])
    ex = jnp.exp(s)

    bt = bt_ref[...].reshape(_BLK, 1)
    seg_ids = jax.lax.broadcasted_iota(jnp.int32, (_BLK, _NUM_SEG), 1)
    ohb = (seg_ids == bt).astype(jnp.bfloat16)

    xe = (x * ex).astype(jnp.bfloat16)
    s_acc[...] += jax.lax.dot_general(
        ohb, xe, (((0,), (0,)), ((), ())),
        preferred_element_type=jnp.float32)
    d_acc[...] += jnp.sum(ohb.astype(jnp.float32) * ex, axis=0,
                          keepdims=True)

    @pl.when(i == _NBLK - 1)
    def _finish():
        inv = 1.0 / (d_acc[...] + 1e-16)
        r = jax.lax.broadcasted_iota(jnp.int32, (_NUM_SEG, _NUM_SEG), 0)
        c = jax.lax.broadcasted_iota(jnp.int32, (_NUM_SEG, _NUM_SEG), 1)
        diag_inv = jnp.where(r == c, inv, 0.0)
        out_ref[...] = jnp.dot(diag_inv, s_acc[...],
                               preferred_element_type=jnp.float32)


@jax.jit
def kernel(x, batch, W1, b1, W2, b2):
    bt3 = batch.astype(jnp.int32).reshape(_NBLK, _BLK, 1)
    b1r = b1.reshape(1, 64).astype(jnp.float32)
    b2r = b2.reshape(1, 1).astype(jnp.float32)
    w2r = W2.reshape(1, 64).astype(jnp.float32)
    out = pl.pallas_call(
        _body,
        grid=(_NBLK,),
        in_specs=[
            pl.BlockSpec((_BLK, _D), lambda i: (i, 0)),
            pl.BlockSpec((1, _BLK, 1), lambda i: (i, 0, 0)),
            pl.BlockSpec((_D, 64), lambda i: (0, 0)),
            pl.BlockSpec((1, 64), lambda i: (0, 0)),
            pl.BlockSpec((1, 64), lambda i: (0, 0)),
            pl.BlockSpec((1, 1), lambda i: (0, 0)),
        ],
        out_specs=pl.BlockSpec((_NUM_SEG, _D), lambda i: (0, 0)),
        out_shape=jax.ShapeDtypeStruct((_NUM_SEG, _D), jnp.float32),
        scratch_shapes=[
            pltpu.VMEM((_NUM_SEG, _D), jnp.float32),
            pltpu.VMEM((1, _NUM_SEG), jnp.float32),
        ],
        compiler_params=pltpu.CompilerParams(
            dimension_semantics=("arbitrary",),
        ),
    )(x, bt3, W1, b1r, w2r, b2r)
    return out


# speedup vs baseline: 2.2400x; 2.2400x over previous
import functools

import jax
import jax.numpy as jnp
from jax.experimental import pallas as pl
from jax.experimental.pallas import tpu as pltpu

_NUM_SEG = 256
_N = 100000
_D = 128
_BLK = 10000
_NBLK = _N // _BLK


def _body(x_ref, bt_ref, w1_ref, b1_ref, w2_ref, b2_ref, out_ref,
          s_acc, d_acc):
    i = pl.program_id(0)

    @pl.when(i == 0)
    def _init():
        s_acc[...] = jnp.zeros_like(s_acc)
        d_acc[...] = jnp.zeros_like(d_acc)

    x = x_ref[...]
    h = jnp.tanh(
        jnp.dot(x, w1_ref[...], preferred_element_type=jnp.float32)
        + b1_ref[...])
    s = (jnp.dot(h, w2_ref[...], preferred_element_type=jnp.float32)
         + b2_ref[...])
    ex = jnp.exp(s)

    bt = bt_ref[...].reshape(_BLK, 1)
    seg_ids = jax.lax.broadcasted_iota(jnp.int32, (_BLK, _NUM_SEG), 1)
    oh = (seg_ids == bt).astype(jnp.float32)

    xe = x * ex
    s_acc[...] += jax.lax.dot_general(
        oh, xe, (((0,), (0,)), ((), ())),
        preferred_element_type=jnp.float32)
    d_acc[...] += jax.lax.dot_general(
        oh, ex, (((0,), (0,)), ((), ())),
        preferred_element_type=jnp.float32)

    @pl.when(i == _NBLK - 1)
    def _finish():
        inv = 1.0 / (d_acc[...] + 1e-16)
        out_ref[...] = s_acc[...] * inv


@jax.jit
def kernel(x, batch, W1, b1, W2, b2):
    bt3 = batch.astype(jnp.int32).reshape(_NBLK, _BLK, 1)
    b1r = b1.reshape(1, 64).astype(jnp.float32)
    b2r = b2.reshape(1, 1).astype(jnp.float32)
    out = pl.pallas_call(
        _body,
        grid=(_NBLK,),
        in_specs=[
            pl.BlockSpec((_BLK, _D), lambda i: (i, 0)),
            pl.BlockSpec((1, _BLK, 1), lambda i: (i, 0, 0)),
            pl.BlockSpec((_D, 64), lambda i: (0, 0)),
            pl.BlockSpec((1, 64), lambda i: (0, 0)),
            pl.BlockSpec((64, 1), lambda i: (0, 0)),
            pl.BlockSpec((1, 1), lambda i: (0, 0)),
        ],
        out_specs=pl.BlockSpec((_NUM_SEG, _D), lambda i: (0, 0)),
        out_shape=jax.ShapeDtypeStruct((_NUM_SEG, _D), jnp.float32),
        scratch_shapes=[
            pltpu.VMEM((_NUM_SEG, _D), jnp.float32),
            pltpu.VMEM((_NUM_SEG, 1), jnp.float32),
        ],
        compiler_params=pltpu.CompilerParams(
            dimension_semantics=("arbitrary",),
        ),
    )(x, bt3, W1, b1r, W2, b2r)
    return out


# speedup vs baseline: 2.2705x; 1.0136x over previous
import jax
import jax.numpy as jnp
from jax.experimental import pallas as pl
from jax.experimental.pallas import tpu as pltpu

_NUM_SEG = 256
_N = 100000
_D = 128
_BLK = 10000
_NBLK = _N // _BLK


def _body(x_ref, bt_ref, w1_ref, b1_ref, w2_ref, b2_ref, out_ref,
          s_acc, d_acc):
    i = pl.program_id(0)

    @pl.when(i == 0)
    def _init():
        s_acc[...] = jnp.zeros_like(s_acc)
        d_acc[...] = jnp.zeros_like(d_acc)

    x = x_ref[...]
    xb = x.astype(jnp.bfloat16)
    h = jnp.tanh(
        jnp.dot(xb, w1_ref[...], preferred_element_type=jnp.float32)
        + b1_ref[...])
    s = (jnp.dot(h.astype(jnp.bfloat16), w2_ref[...],
                 preferred_element_type=jnp.float32)
         + b2_ref[...])
    ex = jnp.exp(s)

    bt = bt_ref[...]
    seg_ids = jax.lax.broadcasted_iota(jnp.int16, (_BLK, _NUM_SEG), 1)
    oh = jnp.where(seg_ids == bt,
                   jnp.bfloat16(1), jnp.bfloat16(0))

    xe = (x * ex).astype(jnp.bfloat16)
    s_acc[...] += jax.lax.dot_general(
        oh, xe, (((0,), (0,)), ((), ())),
        preferred_element_type=jnp.float32)
    d_acc[...] += jax.lax.dot_general(
        oh, ex.astype(jnp.bfloat16), (((0,), (0,)), ((), ())),
        preferred_element_type=jnp.float32)

    @pl.when(i == _NBLK - 1)
    def _finish():
        inv = 1.0 / (d_acc[...] + 1e-16)
        out_ref[...] = s_acc[...] * inv


@jax.jit
def kernel(x, batch, W1, b1, W2, b2):
    bt2 = batch.astype(jnp.int16).reshape(_N, 1)
    b1r = b1.reshape(1, 64).astype(jnp.float32)
    b2r = b2.reshape(1, 1).astype(jnp.float32)
    w1b = W1.astype(jnp.bfloat16)
    w2b = W2.astype(jnp.bfloat16)
    out = pl.pallas_call(
        _body,
        grid=(_NBLK,),
        in_specs=[
            pl.BlockSpec((_BLK, _D), lambda i: (i, 0)),
            pl.BlockSpec((_BLK, 1), lambda i: (i, 0)),
            pl.BlockSpec((_D, 64), lambda i: (0, 0)),
            pl.BlockSpec((1, 64), lambda i: (0, 0)),
            pl.BlockSpec((64, 1), lambda i: (0, 0)),
            pl.BlockSpec((1, 1), lambda i: (0, 0)),
        ],
        out_specs=pl.BlockSpec((_NUM_SEG, _D), lambda i: (0, 0)),
        out_shape=jax.ShapeDtypeStruct((_NUM_SEG, _D), jnp.float32),
        scratch_shapes=[
            pltpu.VMEM((_NUM_SEG, _D), jnp.float32),
            pltpu.VMEM((_NUM_SEG, 1), jnp.float32),
        ],
        compiler_params=pltpu.CompilerParams(
            dimension_semantics=("arbitrary",),
        ),
    )(x, bt2, w1b, b1r, w2b, b2r)
    return out
